# Initial kernel scaffold; baseline (speedup 1.0000x reference)
import jax
import jax.numpy as jnp
from jax.experimental import pallas as pl


def kernel(x0, x1, x2, W_self0, W_neigh0, b0, W_self1, W_neigh1, b1, src0, dst0, src1, dst1, rb2):
    raise NotImplementedError("write your pallas kernel here")


# speedup vs baseline: 4.3110x; 4.3110x over previous
import dataclasses

import jax
import jax.numpy as jnp
from jax import lax
from jax.experimental import pallas as pl
from jax.experimental.pallas import tpu as pltpu
from jax.experimental.pallas import tpu_sc as plsc

N0, N1, N2 = 100000, 25000, 1024
E0, E1 = 400000, 16384
D = 128
HALF = N1 // 2
NC, NS = 2, 16
B = 128
HB = B // 2
CH = 56
CW = 128
CH2 = 8
ROWS_PER_SUB = 784
ACC0 = NS * ROWS_PER_SUB
DUMMY = HALF

_sc_params = pltpu.CompilerParams()
if "needs_layout_passes" in pltpu.CompilerParams.__dataclass_fields__:
    _sc_params = dataclasses.replace(_sc_params, needs_layout_passes=False)

_mesh = plsc.VectorSubcoreMesh(
    core_axis_name="c", subcore_axis_name="s", num_cores=NC, num_subcores=NS
)


def _split_body(dst_ref, out_ref):
    d = dst_ref[...]
    s = jnp.sum(jnp.where(d < HALF, 1, 0).astype(jnp.int32))
    out_ref[...] = jnp.full((8, 128), s, jnp.int32)


def _split_call(dst0):
    return pl.pallas_call(
        _split_body,
        out_shape=jax.ShapeDtypeStruct((8, 128), jnp.int32),
    )(dst0.reshape(3125, 128))


def _sc_layer0_body(x0_hbm, src_hbm, dst_hbm, split_hbm, zacc_hbm, seg_out,
                    split_v, dst_v, src_a, src_b, idx_a, idx_b, rows_v,
                    acc_sh):
    c = lax.axis_index("c")
    s = lax.axis_index("s")

    pltpu.sync_copy(split_hbm, split_v)
    sp = jnp.max(split_v[0, pl.ds(0, 16)])

    pltpu.sync_copy(zacc_hbm.at[pl.ds(0, CH)], rows_v.at[pl.ds(0, CH)])
    for i in range(ROWS_PER_SUB // CH):
        r = s * ROWS_PER_SUB + i * CH
        pltpu.sync_copy(rows_v.at[pl.ds(0, CH)], acc_sh.at[pl.ds(r, CH)])
    plsc.subcore_barrier()

    lo = c * sp
    hi = sp + c * (E0 - sp)
    chunk = (hi - lo + NS - 1) // NS
    my_lo = jnp.minimum(lo + s * chunk, hi)
    my_hi = jnp.minimum(my_lo + chunk, hi)
    a0 = (my_lo // 8) * 8
    nblk = (my_hi - a0 + B - 1) // B
    base = c * HALF
    lane = lax.iota(jnp.int32, 16)

    def _block(k, carry):
        e0 = a0 + k * B
        pltpu.sync_copy(dst_hbm.at[pl.ds(e0, B)], dst_v)
        pltpu.sync_copy(src_hbm.at[pl.ds(e0, HB)], src_a)
        pltpu.sync_copy(src_hbm.at[pl.ds(e0 + HB, HB)], src_b)
        for j in range(B // 16):
            gi = lane + (e0 + j * 16)
            d = dst_v[pl.ds(j * 16, 16)]
            valid = (gi >= my_lo) & (gi < my_hi)
            idx = jnp.where(valid, d - base, DUMMY)
            if j < B // 32:
                idx_a[pl.ds(j * 16, 16)] = idx
            else:
                idx_b[pl.ds(j * 16 - HB, 16)] = idx
        for src_h, idx_h in ((src_a, idx_a), (src_b, idx_b)):
            pltpu.sync_copy(x0_hbm.at[src_h], rows_v)
            pltpu.sync_copy(rows_v, acc_sh.at[idx_h], add=True)
        return carry

    lax.fori_loop(0, nblk, _block, 0)
    plsc.subcore_barrier()

    for i in range(ROWS_PER_SUB // CH):
        r = s * ROWS_PER_SUB + i * CH
        o = c * ACC0 + r
        pltpu.sync_copy(acc_sh.at[pl.ds(r, CH)], rows_v.at[pl.ds(0, CH)])
        pltpu.sync_copy(rows_v.at[pl.ds(0, CH)], seg_out.at[pl.ds(o, CH)])


def _sc_layer0(x0, src0p, dst0p, split, zacc):
    return pl.kernel(
        _sc_layer0_body,
        out_type=jax.ShapeDtypeStruct((NC * ACC0, D), jnp.float32),
        mesh=_mesh,
        scratch_types=[
            pltpu.VMEM((8, 128), jnp.int32),
            pltpu.VMEM((B,), jnp.int32),
            pltpu.VMEM((HB,), jnp.int32),
            pltpu.VMEM((HB,), jnp.int32),
            pltpu.VMEM((HB,), jnp.int32),
            pltpu.VMEM((HB,), jnp.int32),
            pltpu.VMEM((HB, D), jnp.float32),
            pltpu.VMEM_SHARED((ACC0, D), jnp.float32),
        ],
        compiler_params=_sc_params,
    )(x0, src0p, dst0p, split, zacc)


def _sc_cnt0_body(dst_hbm, split_hbm, zcnt_hbm, ones_hbm, cnt_out,
                  split_v, dst_v, idx_a, idx_b, ones_v, zcnt_v, cnt_sh):
    c = lax.axis_index("c")
    s = lax.axis_index("s")

    pltpu.sync_copy(split_hbm, split_v)
    sp = jnp.max(split_v[0, pl.ds(0, 16)])

    pltpu.sync_copy(zcnt_hbm.at[pl.ds(0, CH2)], zcnt_v)
    pltpu.sync_copy(ones_hbm.at[pl.ds(0, HB)], ones_v)
    for i in range(ROWS_PER_SUB // CH2):
        r = s * ROWS_PER_SUB + i * CH2
        pltpu.sync_copy(zcnt_v, cnt_sh.at[pl.ds(r, CH2)])
    plsc.subcore_barrier()

    lo = c * sp
    hi = sp + c * (E0 - sp)
    chunk = (hi - lo + NS - 1) // NS
    my_lo = jnp.minimum(lo + s * chunk, hi)
    my_hi = jnp.minimum(my_lo + chunk, hi)
    a0 = (my_lo // 8) * 8
    nblk = (my_hi - a0 + B - 1) // B
    base = c * HALF
    lane = lax.iota(jnp.int32, 16)

    def _block(k, carry):
        e0 = a0 + k * B
        pltpu.sync_copy(dst_hbm.at[pl.ds(e0, B)], dst_v)
        for j in range(B // 16):
            gi = lane + (e0 + j * 16)
            d = dst_v[pl.ds(j * 16, 16)]
            valid = (gi >= my_lo) & (gi < my_hi)
            idx = jnp.where(valid, d - base, DUMMY)
            if j < B // 32:
                idx_a[pl.ds(j * 16, 16)] = idx
            else:
                idx_b[pl.ds(j * 16 - HB, 16)] = idx
        pltpu.sync_copy(ones_v, cnt_sh.at[idx_a], add=True)
        pltpu.sync_copy(ones_v, cnt_sh.at[idx_b], add=True)
        return carry

    lax.fori_loop(0, nblk, _block, 0)
    plsc.subcore_barrier()

    for i in range(ROWS_PER_SUB // CH2):
        r = s * ROWS_PER_SUB + i * CH2
        o = c * ACC0 + r
        pltpu.sync_copy(cnt_sh.at[pl.ds(r, CH2)], zcnt_v)
        pltpu.sync_copy(zcnt_v, cnt_out.at[pl.ds(o, CH2)])


def _sc_cnt0(dst0p, split, zcnt, ones):
    return pl.kernel(
        _sc_cnt0_body,
        out_type=jax.ShapeDtypeStruct((NC * ACC0, CW), jnp.float32),
        mesh=_mesh,
        scratch_types=[
            pltpu.VMEM((8, 128), jnp.int32),
            pltpu.VMEM((B,), jnp.int32),
            pltpu.VMEM((HB,), jnp.int32),
            pltpu.VMEM((HB,), jnp.int32),
            pltpu.VMEM((HB, CW), jnp.float32),
            pltpu.VMEM((CH2, CW), jnp.float32),
            pltpu.VMEM_SHARED((ACC0, CW), jnp.float32),
        ],
        compiler_params=_sc_params,
    )(dst0p, split, zcnt, ones)


def _tc_layer0_body(x_ref, seg_ref, cnt_ref, ws_ref, wn_ref, b_ref, out_ref):
    x = x_ref[...].reshape(x_ref.shape[1], D)
    seg = seg_ref[...].reshape(seg_ref.shape[1], D)
    cnt = cnt_ref[...].reshape(cnt_ref.shape[1], CW)[:, 0:1]
    hn = seg * (1.0 / jnp.maximum(cnt, 1.0))
    h = (jnp.dot(x, ws_ref[...], preferred_element_type=jnp.float32)
         + jnp.dot(hn, wn_ref[...], preferred_element_type=jnp.float32)
         + b_ref[...])
    out_ref[...] = jnp.maximum(h, 0.0).reshape(out_ref.shape)


def _tc_layer0(x1_3d, seg_3d, cnt_3d, ws, wn, b_2d):
    R = 512
    grid = (2, (N1 // 2 + R - 1) // R)
    return pl.pallas_call(
        _tc_layer0_body,
        grid=grid,
        in_specs=[
            pl.BlockSpec((1, R, D), lambda c, j: (c, j, 0)),
            pl.BlockSpec((1, R, D), lambda c, j: (c, j, 0)),
            pl.BlockSpec((1, R, CW), lambda c, j: (c, j, 0)),
            pl.BlockSpec((D, D), lambda c, j: (0, 0)),
            pl.BlockSpec((D, D), lambda c, j: (0, 0)),
            pl.BlockSpec((1, D), lambda c, j: (0, 0)),
        ],
        out_specs=pl.BlockSpec((1, R, D), lambda c, j: (c, j, 0)),
        out_shape=jax.ShapeDtypeStruct((2, N1 // 2, D), jnp.float32),
    )(x1_3d, seg_3d, cnt_3d, ws, wn, b_2d)


def _sc_layer1_body(h1_hbm, src_hbm, dst_hbm, zrows_hbm, ones_hbm,
                    part_out, cntp_out,
                    dst_v, src_v, rows_v, ones_v, zcnt_v, acc_sh, cnt_sh):
    c = lax.axis_index("c")
    s = lax.axis_index("s")
    rows = N2 // NS

    pltpu.sync_copy(zrows_hbm, rows_v.at[pl.ds(0, rows)])
    pltpu.sync_copy(zrows_hbm, zcnt_v)
    pltpu.sync_copy(rows_v.at[pl.ds(0, rows)], acc_sh.at[pl.ds(s * rows, rows)])
    pltpu.sync_copy(zcnt_v, cnt_sh.at[pl.ds(s * rows, rows)])
    pltpu.sync_copy(ones_hbm, ones_v)
    plsc.subcore_barrier()

    w = c * NS + s
    for blk in range(E1 // (NC * NS) // B):
        e0 = w * (E1 // (NC * NS)) + blk * B
        pltpu.sync_copy(dst_hbm.at[pl.ds(e0, B)], dst_v)
        pltpu.sync_copy(src_hbm.at[pl.ds(e0, B)], src_v)
        pltpu.sync_copy(h1_hbm.at[src_v], rows_v)
        pltpu.sync_copy(rows_v, acc_sh.at[dst_v], add=True)
        pltpu.sync_copy(ones_v, cnt_sh.at[dst_v], add=True)
    plsc.subcore_barrier()

    r0 = s * rows
    o0 = c * N2 + r0
    pltpu.sync_copy(acc_sh.at[pl.ds(r0, rows)], rows_v.at[pl.ds(0, rows)])
    pltpu.sync_copy(rows_v.at[pl.ds(0, rows)], part_out.at[pl.ds(o0, rows)])
    pltpu.sync_copy(cnt_sh.at[pl.ds(r0, rows)], zcnt_v)
    pltpu.sync_copy(zcnt_v, cntp_out.at[pl.ds(o0, rows)])


def _sc_layer1(h1, src1, dst1, zrows, onesw):
    return pl.kernel(
        _sc_layer1_body,
        out_type=[
            jax.ShapeDtypeStruct((NC * N2, D), jnp.float32),
            jax.ShapeDtypeStruct((NC * N2, CW), jnp.float32),
        ],
        mesh=_mesh,
        scratch_types=[
            pltpu.VMEM((B,), jnp.int32),
            pltpu.VMEM((B,), jnp.int32),
            pltpu.VMEM((B, D), jnp.float32),
            pltpu.VMEM((B, CW), jnp.float32),
            pltpu.VMEM((N2 // NS, CW), jnp.float32),
            pltpu.VMEM_SHARED((N2, D), jnp.float32),
            pltpu.VMEM_SHARED((N2, CW), jnp.float32),
        ],
        compiler_params=_sc_params,
    )(h1, src1, dst1, zrows, onesw)


def _tc_layer1_body(hd_ref, part_ref, cntp_ref, ws_ref, wn_ref, b_ref, out_ref):
    seg = part_ref[pl.ds(0, N2), :] + part_ref[pl.ds(N2, N2), :]
    cnt = (cntp_ref[pl.ds(0, N2), :] + cntp_ref[pl.ds(N2, N2), :])[:, 0:1]
    hn = seg * (1.0 / jnp.maximum(cnt, 1.0))
    out_ref[...] = (jnp.dot(hd_ref[...], ws_ref[...],
                            preferred_element_type=jnp.float32)
                    + jnp.dot(hn, wn_ref[...],
                              preferred_element_type=jnp.float32)
                    + b_ref[...])


def _tc_layer1(hd, part, cntp, ws_p, wn_p, b_2d):
    return pl.pallas_call(
        _tc_layer1_body,
        out_shape=jax.ShapeDtypeStruct((N2, D), jnp.float32),
    )(hd, part, cntp, ws_p, wn_p, b_2d)


def kernel(x0, x1, x2, W_self0, W_neigh0, b0, W_self1, W_neigh1, b1,
           src0, dst0, src1, dst1, rb2):
    split = _split_call(dst0)
    pad = jnp.zeros((B,), jnp.int32)
    src0p = jnp.concatenate([src0, pad])
    dst0p = jnp.concatenate([dst0, pad])
    zrows = jnp.zeros((N2 // NS, D), jnp.float32)
    onesw = jnp.ones((B, CW), jnp.float32)

    seg = _sc_layer0(x0, src0p, dst0p, split, zrows)
    cnt = _sc_cnt0(dst0p, split, zrows, onesw)
    h1 = _tc_layer0(
        x1.reshape(2, N1 // 2, D),
        seg.reshape(2, ACC0, D),
        cnt.reshape(2, ACC0, CW),
        W_self0, W_neigh0, b0.reshape(1, D),
    ).reshape(N1, D)

    part, cntp = _sc_layer1(h1, src1, dst1, zrows, onesw)
    hd = lax.dynamic_slice_in_dim(h1, rb2 - N2, N2)
    ws1p = jnp.pad(W_self1, ((0, 0), (0, D - 40)))
    wn1p = jnp.pad(W_neigh1, ((0, 0), (0, D - 40)))
    b1p = jnp.pad(b1, (0, D - 40)).reshape(1, D)
    outp = _tc_layer1(hd, part, cntp, ws1p, wn1p, b1p)
    return outp[:, :40]


# speedup vs baseline: 4.8615x; 1.1277x over previous
import dataclasses

import jax
import jax.numpy as jnp
from jax import lax
from jax.experimental import pallas as pl
from jax.experimental.pallas import tpu as pltpu
from jax.experimental.pallas import tpu_sc as plsc

N0, N1, N2 = 100000, 25000, 1024
E0, E1 = 400000, 16384
D = 128
HALF = N1 // 2
NC, NS = 2, 16
B = 128
HB = B // 2
CH = 56
CW = 128
CH2 = 8
ROWS_PER_SUB = 784
ACC0 = NS * ROWS_PER_SUB
DUMMY = HALF

_sc_params = pltpu.CompilerParams()
if "needs_layout_passes" in pltpu.CompilerParams.__dataclass_fields__:
    _sc_params = dataclasses.replace(_sc_params, needs_layout_passes=False)

_mesh = plsc.VectorSubcoreMesh(
    core_axis_name="c", subcore_axis_name="s", num_cores=NC, num_subcores=NS
)


def _split_body(dst_ref, out_ref):
    d = dst_ref[...]
    s = jnp.sum(jnp.where(d < HALF, 1, 0).astype(jnp.int32))
    out_ref[...] = jnp.full((8, 16), s, jnp.int32)


def _split_call(dst0):
    return pl.pallas_call(
        _split_body,
        out_shape=jax.ShapeDtypeStruct((8, 16), jnp.int32),
    )(dst0.reshape(3125, 128))


def _sc_layer0_body(x0_hbm, src_hbm, dst_hbm, split_hbm, zacc_hbm, seg_out,
                    split_v, dst_v, src_a, src_b, idx_a, idx_b, rows_a, rows_b,
                    sem_ga, sem_gb, sem_sa, sem_sb, acc_sh):
    c = lax.axis_index("c")
    s = lax.axis_index("s")

    pltpu.sync_copy(split_hbm, split_v)
    sp = jnp.max(split_v[0, pl.ds(0, 16)])

    pltpu.sync_copy(zacc_hbm.at[pl.ds(0, CH)], rows_a.at[pl.ds(0, CH)])
    for i in range(ROWS_PER_SUB // CH):
        r = s * ROWS_PER_SUB + i * CH
        pltpu.sync_copy(rows_a.at[pl.ds(0, CH)], acc_sh.at[pl.ds(r, CH)])
    plsc.subcore_barrier()

    lo = c * sp
    hi = sp + c * (E0 - sp)
    chunk = (hi - lo + NS - 1) // NS
    my_lo = jnp.minimum(lo + s * chunk, hi)
    my_hi = jnp.minimum(my_lo + chunk, hi)
    a0 = (my_lo // 8) * 8
    nblk = (my_hi - a0 + B - 1) // B
    base = c * HALF
    lane = lax.iota(jnp.int32, 16)

    def _block(k, carry):
        e0 = a0 + k * B
        pltpu.sync_copy(dst_hbm.at[pl.ds(e0, B)], dst_v)
        pltpu.sync_copy(src_hbm.at[pl.ds(e0, HB)], src_a)
        pltpu.sync_copy(src_hbm.at[pl.ds(e0 + HB, HB)], src_b)
        for j in range(B // 16):
            gi = lane + (e0 + j * 16)
            d = dst_v[pl.ds(j * 16, 16)]
            valid = (gi >= my_lo) & (gi < my_hi)
            idx = jnp.where(valid, d - base, DUMMY)
            if j < B // 32:
                idx_a[pl.ds(j * 16, 16)] = idx
            else:
                idx_b[pl.ds(j * 16 - HB, 16)] = idx
        ga = pltpu.async_copy(x0_hbm.at[src_a], rows_a, sem_ga)
        gb = pltpu.async_copy(x0_hbm.at[src_b], rows_b, sem_gb)
        ga.wait()
        sa = pltpu.async_copy(rows_a, acc_sh.at[idx_a], sem_sa, add=True)
        gb.wait()
        sb = pltpu.async_copy(rows_b, acc_sh.at[idx_b], sem_sb, add=True)
        sa.wait()
        sb.wait()
        return carry

    lax.fori_loop(0, nblk, _block, 0)
    plsc.subcore_barrier()

    for i in range(ROWS_PER_SUB // CH):
        r = s * ROWS_PER_SUB + i * CH
        o = c * ACC0 + r
        pltpu.sync_copy(acc_sh.at[pl.ds(r, CH)], rows_a.at[pl.ds(0, CH)])
        pltpu.sync_copy(rows_a.at[pl.ds(0, CH)], seg_out.at[pl.ds(o, CH)])


def _sc_layer0(x0, src0p, dst0p, split, zacc):
    return pl.kernel(
        _sc_layer0_body,
        out_type=jax.ShapeDtypeStruct((NC * ACC0, D), jnp.float32),
        mesh=_mesh,
        scratch_types=[
            pltpu.VMEM((8, 16), jnp.int32),
            pltpu.VMEM((B,), jnp.int32),
            pltpu.VMEM((HB,), jnp.int32),
            pltpu.VMEM((HB,), jnp.int32),
            pltpu.VMEM((HB,), jnp.int32),
            pltpu.VMEM((HB,), jnp.int32),
            pltpu.VMEM((HB, D), jnp.float32),
            pltpu.VMEM((HB, D), jnp.float32),
            pltpu.SemaphoreType.DMA,
            pltpu.SemaphoreType.DMA,
            pltpu.SemaphoreType.DMA,
            pltpu.SemaphoreType.DMA,
            pltpu.VMEM_SHARED((ACC0, D), jnp.float32),
        ],
        compiler_params=_sc_params,
    )(x0, src0p, dst0p, split, zacc)


def _sc_cnt0_body(dst_hbm, split_hbm, zcnt_hbm, ones_hbm, cnt_out,
                  split_v, dst_v, idx_a, idx_b, ones_v, zcnt_v, cnt_sh):
    c = lax.axis_index("c")
    s = lax.axis_index("s")

    pltpu.sync_copy(split_hbm, split_v)
    sp = jnp.max(split_v[0, pl.ds(0, 16)])

    pltpu.sync_copy(zcnt_hbm.at[pl.ds(0, CH2)], zcnt_v)
    pltpu.sync_copy(ones_hbm.at[pl.ds(0, HB)], ones_v)
    for i in range(ROWS_PER_SUB // CH2):
        r = s * ROWS_PER_SUB + i * CH2
        pltpu.sync_copy(zcnt_v, cnt_sh.at[pl.ds(r, CH2)])
    plsc.subcore_barrier()

    lo = c * sp
    hi = sp + c * (E0 - sp)
    chunk = (hi - lo + NS - 1) // NS
    my_lo = jnp.minimum(lo + s * chunk, hi)
    my_hi = jnp.minimum(my_lo + chunk, hi)
    a0 = (my_lo // 8) * 8
    nblk = (my_hi - a0 + B - 1) // B
    base = c * HALF
    lane = lax.iota(jnp.int32, 16)

    def _block(k, carry):
        e0 = a0 + k * B
        pltpu.sync_copy(dst_hbm.at[pl.ds(e0, B)], dst_v)
        for j in range(B // 16):
            gi = lane + (e0 + j * 16)
            d = dst_v[pl.ds(j * 16, 16)]
            valid = (gi >= my_lo) & (gi < my_hi)
            idx = jnp.where(valid, d - base, DUMMY)
            if j < B // 32:
                idx_a[pl.ds(j * 16, 16)] = idx
            else:
                idx_b[pl.ds(j * 16 - HB, 16)] = idx
        pltpu.sync_copy(ones_v, cnt_sh.at[idx_a], add=True)
        pltpu.sync_copy(ones_v, cnt_sh.at[idx_b], add=True)
        return carry

    lax.fori_loop(0, nblk, _block, 0)
    plsc.subcore_barrier()

    for i in range(ROWS_PER_SUB // CH2):
        r = s * ROWS_PER_SUB + i * CH2
        o = c * ACC0 + r
        pltpu.sync_copy(cnt_sh.at[pl.ds(r, CH2)], zcnt_v)
        pltpu.sync_copy(zcnt_v, cnt_out.at[pl.ds(o, CH2)])


def _sc_cnt0(dst0p, split, zcnt, ones):
    return pl.kernel(
        _sc_cnt0_body,
        out_type=jax.ShapeDtypeStruct((NC * ACC0, CW), jnp.float32),
        mesh=_mesh,
        scratch_types=[
            pltpu.VMEM((8, 16), jnp.int32),
            pltpu.VMEM((B,), jnp.int32),
            pltpu.VMEM((HB,), jnp.int32),
            pltpu.VMEM((HB,), jnp.int32),
            pltpu.VMEM((HB, CW), jnp.float32),
            pltpu.VMEM((CH2, CW), jnp.float32),
            pltpu.VMEM_SHARED((ACC0, CW), jnp.float32),
        ],
        compiler_params=_sc_params,
    )(dst0p, split, zcnt, ones)


def _tc_layer0_body(x_ref, seg_ref, cnt_ref, ws_ref, wn_ref, b_ref, out_ref):
    x = x_ref[...].reshape(x_ref.shape[1], D)
    seg = seg_ref[...].reshape(seg_ref.shape[1], D)
    cnt = cnt_ref[...].reshape(cnt_ref.shape[1], CW)[:, 0:1]
    hn = seg * (1.0 / jnp.maximum(cnt, 1.0))
    h = (jnp.dot(x, ws_ref[...], preferred_element_type=jnp.float32)
         + jnp.dot(hn, wn_ref[...], preferred_element_type=jnp.float32)
         + b_ref[...])
    out_ref[...] = jnp.maximum(h, 0.0).reshape(out_ref.shape)


def _tc_layer0(x1_3d, seg_3d, cnt_3d, ws, wn, b_2d):
    R = 512
    grid = (2, (N1 // 2 + R - 1) // R)
    return pl.pallas_call(
        _tc_layer0_body,
        grid=grid,
        in_specs=[
            pl.BlockSpec((1, R, D), lambda c, j: (c, j, 0)),
            pl.BlockSpec((1, R, D), lambda c, j: (c, j, 0)),
            pl.BlockSpec((1, R, CW), lambda c, j: (c, j, 0)),
            pl.BlockSpec((D, D), lambda c, j: (0, 0)),
            pl.BlockSpec((D, D), lambda c, j: (0, 0)),
            pl.BlockSpec((1, D), lambda c, j: (0, 0)),
        ],
        out_specs=pl.BlockSpec((1, R, D), lambda c, j: (c, j, 0)),
        out_shape=jax.ShapeDtypeStruct((2, N1 // 2, D), jnp.float32),
    )(x1_3d, seg_3d, cnt_3d, ws, wn, b_2d)


def _sc_layer1_body(h1_hbm, src_hbm, dst_hbm, zrows_hbm, ones_hbm,
                    part_out, cntp_out,
                    dst_v, src_v, rows_v, ones_v, zcnt_v, acc_sh, cnt_sh):
    c = lax.axis_index("c")
    s = lax.axis_index("s")
    rows = N2 // NS

    pltpu.sync_copy(zrows_hbm, rows_v.at[pl.ds(0, rows)])
    pltpu.sync_copy(zrows_hbm, zcnt_v)
    pltpu.sync_copy(rows_v.at[pl.ds(0, rows)], acc_sh.at[pl.ds(s * rows, rows)])
    pltpu.sync_copy(zcnt_v, cnt_sh.at[pl.ds(s * rows, rows)])
    pltpu.sync_copy(ones_hbm, ones_v)
    plsc.subcore_barrier()

    w = c * NS + s
    for blk in range(E1 // (NC * NS) // B):
        e0 = w * (E1 // (NC * NS)) + blk * B
        pltpu.sync_copy(dst_hbm.at[pl.ds(e0, B)], dst_v)
        pltpu.sync_copy(src_hbm.at[pl.ds(e0, B)], src_v)
        pltpu.sync_copy(h1_hbm.at[src_v], rows_v)
        pltpu.sync_copy(rows_v, acc_sh.at[dst_v], add=True)
        pltpu.sync_copy(ones_v, cnt_sh.at[dst_v], add=True)
    plsc.subcore_barrier()

    r0 = s * rows
    o0 = c * N2 + r0
    pltpu.sync_copy(acc_sh.at[pl.ds(r0, rows)], rows_v.at[pl.ds(0, rows)])
    pltpu.sync_copy(rows_v.at[pl.ds(0, rows)], part_out.at[pl.ds(o0, rows)])
    pltpu.sync_copy(cnt_sh.at[pl.ds(r0, rows)], zcnt_v)
    pltpu.sync_copy(zcnt_v, cntp_out.at[pl.ds(o0, rows)])


def _sc_layer1(h1, src1, dst1, zrows, onesw):
    return pl.kernel(
        _sc_layer1_body,
        out_type=[
            jax.ShapeDtypeStruct((NC * N2, D), jnp.float32),
            jax.ShapeDtypeStruct((NC * N2, CW), jnp.float32),
        ],
        mesh=_mesh,
        scratch_types=[
            pltpu.VMEM((B,), jnp.int32),
            pltpu.VMEM((B,), jnp.int32),
            pltpu.VMEM((B, D), jnp.float32),
            pltpu.VMEM((B, CW), jnp.float32),
            pltpu.VMEM((N2 // NS, CW), jnp.float32),
            pltpu.VMEM_SHARED((N2, D), jnp.float32),
            pltpu.VMEM_SHARED((N2, CW), jnp.float32),
        ],
        compiler_params=_sc_params,
    )(h1, src1, dst1, zrows, onesw)


def _tc_layer1_body(hd_ref, part_ref, cntp_ref, ws_ref, wn_ref, b_ref, out_ref):
    seg = part_ref[pl.ds(0, N2), :] + part_ref[pl.ds(N2, N2), :]
    cnt = (cntp_ref[pl.ds(0, N2), :] + cntp_ref[pl.ds(N2, N2), :])[:, 0:1]
    hn = seg * (1.0 / jnp.maximum(cnt, 1.0))
    out_ref[...] = (jnp.dot(hd_ref[...], ws_ref[...],
                            preferred_element_type=jnp.float32)
                    + jnp.dot(hn, wn_ref[...],
                              preferred_element_type=jnp.float32)
                    + b_ref[...])


def _tc_layer1(hd, part, cntp, ws_p, wn_p, b_2d):
    return pl.pallas_call(
        _tc_layer1_body,
        out_shape=jax.ShapeDtypeStruct((N2, D), jnp.float32),
    )(hd, part, cntp, ws_p, wn_p, b_2d)


def kernel(x0, x1, x2, W_self0, W_neigh0, b0, W_self1, W_neigh1, b1,
           src0, dst0, src1, dst1, rb2):
    split = _split_call(dst0)
    pad = jnp.zeros((B,), jnp.int32)
    src0p = jnp.concatenate([src0, pad])
    dst0p = jnp.concatenate([dst0, pad])
    zrows = jnp.zeros((N2 // NS, D), jnp.float32)
    onesw = jnp.ones((B, CW), jnp.float32)

    seg = _sc_layer0(x0, src0p, dst0p, split, zrows)
    cnt = _sc_cnt0(dst0p, split, zrows, onesw)
    h1 = _tc_layer0(
        x1.reshape(2, N1 // 2, D),
        seg.reshape(2, ACC0, D),
        cnt.reshape(2, ACC0, CW),
        W_self0, W_neigh0, b0.reshape(1, D),
    ).reshape(N1, D)

    part, cntp = _sc_layer1(h1, src1, dst1, zrows, onesw)
    hd = lax.dynamic_slice_in_dim(h1, rb2 - N2, N2)
    ws1p = jnp.pad(W_self1, ((0, 0), (0, D - 40)))
    wn1p = jnp.pad(W_neigh1, ((0, 0), (0, D - 40)))
    b1p = jnp.pad(b1, (0, D - 40)).reshape(1, D)
    outp = _tc_layer1(hd, part, cntp, ws1p, wn1p, b1p)
    return outp[:, :40]


# speedup vs baseline: 6.4510x; 1.3270x over previous
import dataclasses

import jax
import jax.numpy as jnp
from jax import lax
from jax.experimental import pallas as pl
from jax.experimental.pallas import tpu as pltpu
from jax.experimental.pallas import tpu_sc as plsc

N0, N1, N2 = 100000, 25000, 1024
E0, E1 = 400000, 16384
D = 128
HALF = N1 // 2
NC, NS = 2, 16
B = 128
HB = B // 2
CH = 56
CW = 128
CH2 = 8
SB = 1024
ROWS_PER_SUB = 784
ACC0 = NS * ROWS_PER_SUB
DUMMY = HALF

_sc_params = pltpu.CompilerParams()
if "needs_layout_passes" in pltpu.CompilerParams.__dataclass_fields__:
    _sc_params = dataclasses.replace(_sc_params, needs_layout_passes=False)

_mesh = plsc.VectorSubcoreMesh(
    core_axis_name="c", subcore_axis_name="s", num_cores=NC, num_subcores=NS
)


def _split_body(dst_ref, out_ref):
    d = dst_ref[...]
    s = jnp.sum(jnp.where(d < HALF, 1, 0).astype(jnp.int32))
    out_ref[...] = jnp.full((8, 16), s, jnp.int32)


def _split_call(dst0):
    return pl.pallas_call(
        _split_body,
        out_shape=jax.ShapeDtypeStruct((8, 16), jnp.int32),
    )(dst0.reshape(3125, 128))


def _sc_layer0_body(x0_hbm, src_hbm, dst_hbm, split_hbm, zacc_hbm, seg_out,
                    split_v, dst_v, src_v, idx_a, idx_b, rows_a, rows_b,
                    sem_ga, sem_gb, sem_sa, sem_sb, acc_sh):
    c = lax.axis_index("c")
    s = lax.axis_index("s")

    pltpu.sync_copy(split_hbm, split_v)
    sp = jnp.max(split_v[0, pl.ds(0, 16)])

    pltpu.sync_copy(zacc_hbm.at[pl.ds(0, CH)], rows_a.at[pl.ds(0, CH)])
    for i in range(ROWS_PER_SUB // CH):
        r = s * ROWS_PER_SUB + i * CH
        pltpu.sync_copy(rows_a.at[pl.ds(0, CH)], acc_sh.at[pl.ds(r, CH)])
    plsc.subcore_barrier()

    lo = c * sp
    hi = sp + c * (E0 - sp)
    chunk = (hi - lo + NS - 1) // NS
    my_lo = jnp.minimum(lo + s * chunk, hi)
    my_hi = jnp.minimum(my_lo + chunk, hi)
    a0 = (my_lo // 8) * 8
    nsb = (my_hi - a0 + SB - 1) // SB
    base = c * HALF
    lane = lax.iota(jnp.int32, 16)

    def _superblock(k, carry):
        e_sb = a0 + k * SB
        pltpu.sync_copy(dst_hbm.at[pl.ds(e_sb, SB)], dst_v)
        pltpu.sync_copy(src_hbm.at[pl.ds(e_sb, SB)], src_v)
        for blk in range(SB // B):
            e0 = e_sb + blk * B

            @pl.when(e0 < my_hi)
            def _do():
                for j in range(B // 16):
                    gi = lane + (e0 + j * 16)
                    d = dst_v[pl.ds(blk * B + j * 16, 16)]
                    valid = (gi >= my_lo) & (gi < my_hi)
                    idx = jnp.where(valid, d - base, DUMMY)
                    if j < B // 32:
                        idx_a[pl.ds(j * 16, 16)] = idx
                    else:
                        idx_b[pl.ds(j * 16 - HB, 16)] = idx
                src_ha = src_v.at[pl.ds(blk * B, HB)]
                src_hb = src_v.at[pl.ds(blk * B + HB, HB)]
                ga = pltpu.async_copy(x0_hbm.at[src_ha], rows_a, sem_ga)
                gb = pltpu.async_copy(x0_hbm.at[src_hb], rows_b, sem_gb)
                ga.wait()
                sa = pltpu.async_copy(rows_a, acc_sh.at[idx_a], sem_sa, add=True)
                gb.wait()
                sb = pltpu.async_copy(rows_b, acc_sh.at[idx_b], sem_sb, add=True)
                sa.wait()
                sb.wait()
        return carry

    lax.fori_loop(0, nsb, _superblock, 0)
    plsc.subcore_barrier()

    for i in range(ROWS_PER_SUB // CH):
        r = s * ROWS_PER_SUB + i * CH
        o = c * ACC0 + r
        pltpu.sync_copy(acc_sh.at[pl.ds(r, CH)], rows_a.at[pl.ds(0, CH)])
        pltpu.sync_copy(rows_a.at[pl.ds(0, CH)], seg_out.at[pl.ds(o, CH)])


def _sc_layer0(x0, src0p, dst0p, split, zacc):
    return pl.kernel(
        _sc_layer0_body,
        out_type=jax.ShapeDtypeStruct((NC * ACC0, D), jnp.float32),
        mesh=_mesh,
        scratch_types=[
            pltpu.VMEM((8, 16), jnp.int32),
            pltpu.VMEM((SB,), jnp.int32),
            pltpu.VMEM((SB,), jnp.int32),
            pltpu.VMEM((HB,), jnp.int32),
            pltpu.VMEM((HB,), jnp.int32),
            pltpu.VMEM((HB, D), jnp.float32),
            pltpu.VMEM((HB, D), jnp.float32),
            pltpu.SemaphoreType.DMA,
            pltpu.SemaphoreType.DMA,
            pltpu.SemaphoreType.DMA,
            pltpu.SemaphoreType.DMA,
            pltpu.VMEM_SHARED((ACC0, D), jnp.float32),
        ],
        compiler_params=_sc_params,
    )(x0, src0p, dst0p, split, zacc)


def _sc_cnt0_body(dst_hbm, split_hbm, zcnt_hbm, ones_hbm, cnt_out,
                  split_v, dst_v, idx_a, idx_b, ones_v, zcnt_v,
                  sem_sa, sem_sb, cnt_sh):
    c = lax.axis_index("c")
    s = lax.axis_index("s")

    pltpu.sync_copy(split_hbm, split_v)
    sp = jnp.max(split_v[0, pl.ds(0, 16)])

    pltpu.sync_copy(zcnt_hbm.at[pl.ds(0, CH2)], zcnt_v)
    pltpu.sync_copy(ones_hbm.at[pl.ds(0, HB)], ones_v)
    for i in range(ROWS_PER_SUB // CH2):
        r = s * ROWS_PER_SUB + i * CH2
        pltpu.sync_copy(zcnt_v, cnt_sh.at[pl.ds(r, CH2)])
    plsc.subcore_barrier()

    lo = c * sp
    hi = sp + c * (E0 - sp)
    chunk = (hi - lo + NS - 1) // NS
    my_lo = jnp.minimum(lo + s * chunk, hi)
    my_hi = jnp.minimum(my_lo + chunk, hi)
    a0 = (my_lo // 8) * 8
    nsb = (my_hi - a0 + SB - 1) // SB
    base = c * HALF
    lane = lax.iota(jnp.int32, 16)

    def _superblock(k, carry):
        e_sb = a0 + k * SB
        pltpu.sync_copy(dst_hbm.at[pl.ds(e_sb, SB)], dst_v)
        for blk in range(SB // B):
            e0 = e_sb + blk * B

            @pl.when(e0 < my_hi)
            def _do():
                for j in range(B // 16):
                    gi = lane + (e0 + j * 16)
                    d = dst_v[pl.ds(blk * B + j * 16, 16)]
                    valid = (gi >= my_lo) & (gi < my_hi)
                    idx = jnp.where(valid, d - base, DUMMY)
                    if j < B // 32:
                        idx_a[pl.ds(j * 16, 16)] = idx
                    else:
                        idx_b[pl.ds(j * 16 - HB, 16)] = idx
                sa = pltpu.async_copy(ones_v, cnt_sh.at[idx_a], sem_sa, add=True)
                sb = pltpu.async_copy(ones_v, cnt_sh.at[idx_b], sem_sb, add=True)
                sa.wait()
                sb.wait()
        return carry

    lax.fori_loop(0, nsb, _superblock, 0)
    plsc.subcore_barrier()

    for i in range(ROWS_PER_SUB // CH2):
        r = s * ROWS_PER_SUB + i * CH2
        o = c * ACC0 + r
        pltpu.sync_copy(cnt_sh.at[pl.ds(r, CH2)], zcnt_v)
        pltpu.sync_copy(zcnt_v, cnt_out.at[pl.ds(o, CH2)])


def _sc_cnt0(dst0p, split, zcnt, ones):
    return pl.kernel(
        _sc_cnt0_body,
        out_type=jax.ShapeDtypeStruct((NC * ACC0, CW), jnp.float32),
        mesh=_mesh,
        scratch_types=[
            pltpu.VMEM((8, 16), jnp.int32),
            pltpu.VMEM((SB,), jnp.int32),
            pltpu.VMEM((HB,), jnp.int32),
            pltpu.VMEM((HB,), jnp.int32),
            pltpu.VMEM((HB, CW), jnp.float32),
            pltpu.VMEM((CH2, CW), jnp.float32),
            pltpu.SemaphoreType.DMA,
            pltpu.SemaphoreType.DMA,
            pltpu.VMEM_SHARED((ACC0, CW), jnp.float32),
        ],
        compiler_params=_sc_params,
    )(dst0p, split, zcnt, ones)


def _tc_layer0_body(x_ref, seg_ref, cnt_ref, ws_ref, wn_ref, b_ref, out_ref):
    x = x_ref[...].reshape(x_ref.shape[1], D)
    seg = seg_ref[...].reshape(seg_ref.shape[1], D)
    cnt = cnt_ref[...].reshape(cnt_ref.shape[1], CW)[:, 0:1]
    hn = seg * (1.0 / jnp.maximum(cnt, 1.0))
    h = (jnp.dot(x, ws_ref[...], preferred_element_type=jnp.float32)
         + jnp.dot(hn, wn_ref[...], preferred_element_type=jnp.float32)
         + b_ref[...])
    out_ref[...] = jnp.maximum(h, 0.0).reshape(out_ref.shape)


def _tc_layer0(x1_3d, seg_3d, cnt_3d, ws, wn, b_2d):
    R = 512
    grid = (2, (N1 // 2 + R - 1) // R)
    return pl.pallas_call(
        _tc_layer0_body,
        grid=grid,
        in_specs=[
            pl.BlockSpec((1, R, D), lambda c, j: (c, j, 0)),
            pl.BlockSpec((1, R, D), lambda c, j: (c, j, 0)),
            pl.BlockSpec((1, R, CW), lambda c, j: (c, j, 0)),
            pl.BlockSpec((D, D), lambda c, j: (0, 0)),
            pl.BlockSpec((D, D), lambda c, j: (0, 0)),
            pl.BlockSpec((1, D), lambda c, j: (0, 0)),
        ],
        out_specs=pl.BlockSpec((1, R, D), lambda c, j: (c, j, 0)),
        out_shape=jax.ShapeDtypeStruct((2, N1 // 2, D), jnp.float32),
    )(x1_3d, seg_3d, cnt_3d, ws, wn, b_2d)


def _sc_layer1_body(h1_hbm, src_hbm, dst_hbm, zrows_hbm, ones_hbm,
                    part_out, cntp_out,
                    dst_v, src_v, rows_v, ones_v, zcnt_v, acc_sh, cnt_sh):
    c = lax.axis_index("c")
    s = lax.axis_index("s")
    rows = N2 // NS

    pltpu.sync_copy(zrows_hbm, rows_v.at[pl.ds(0, rows)])
    pltpu.sync_copy(zrows_hbm, zcnt_v)
    pltpu.sync_copy(rows_v.at[pl.ds(0, rows)], acc_sh.at[pl.ds(s * rows, rows)])
    pltpu.sync_copy(zcnt_v, cnt_sh.at[pl.ds(s * rows, rows)])
    pltpu.sync_copy(ones_hbm, ones_v)
    plsc.subcore_barrier()

    w = c * NS + s
    for blk in range(E1 // (NC * NS) // B):
        e0 = w * (E1 // (NC * NS)) + blk * B
        pltpu.sync_copy(dst_hbm.at[pl.ds(e0, B)], dst_v)
        pltpu.sync_copy(src_hbm.at[pl.ds(e0, B)], src_v)
        pltpu.sync_copy(h1_hbm.at[src_v], rows_v)
        pltpu.sync_copy(rows_v, acc_sh.at[dst_v], add=True)
        pltpu.sync_copy(ones_v, cnt_sh.at[dst_v], add=True)
    plsc.subcore_barrier()

    r0 = s * rows
    o0 = c * N2 + r0
    pltpu.sync_copy(acc_sh.at[pl.ds(r0, rows)], rows_v.at[pl.ds(0, rows)])
    pltpu.sync_copy(rows_v.at[pl.ds(0, rows)], part_out.at[pl.ds(o0, rows)])
    pltpu.sync_copy(cnt_sh.at[pl.ds(r0, rows)], zcnt_v)
    pltpu.sync_copy(zcnt_v, cntp_out.at[pl.ds(o0, rows)])


def _sc_layer1(h1, src1, dst1, zrows, onesw):
    return pl.kernel(
        _sc_layer1_body,
        out_type=[
            jax.ShapeDtypeStruct((NC * N2, D), jnp.float32),
            jax.ShapeDtypeStruct((NC * N2, CW), jnp.float32),
        ],
        mesh=_mesh,
        scratch_types=[
            pltpu.VMEM((B,), jnp.int32),
            pltpu.VMEM((B,), jnp.int32),
            pltpu.VMEM((B, D), jnp.float32),
            pltpu.VMEM((B, CW), jnp.float32),
            pltpu.VMEM((N2 // NS, CW), jnp.float32),
            pltpu.VMEM_SHARED((N2, D), jnp.float32),
            pltpu.VMEM_SHARED((N2, CW), jnp.float32),
        ],
        compiler_params=_sc_params,
    )(h1, src1, dst1, zrows, onesw)


def _tc_layer1_body(hd_ref, part_ref, cntp_ref, ws_ref, wn_ref, b_ref, out_ref):
    seg = part_ref[pl.ds(0, N2), :] + part_ref[pl.ds(N2, N2), :]
    cnt = (cntp_ref[pl.ds(0, N2), :] + cntp_ref[pl.ds(N2, N2), :])[:, 0:1]
    hn = seg * (1.0 / jnp.maximum(cnt, 1.0))
    out_ref[...] = (jnp.dot(hd_ref[...], ws_ref[...],
                            preferred_element_type=jnp.float32)
                    + jnp.dot(hn, wn_ref[...],
                              preferred_element_type=jnp.float32)
                    + b_ref[...])


def _tc_layer1(hd, part, cntp, ws_p, wn_p, b_2d):
    return pl.pallas_call(
        _tc_layer1_body,
        out_shape=jax.ShapeDtypeStruct((N2, D), jnp.float32),
    )(hd, part, cntp, ws_p, wn_p, b_2d)


def kernel(x0, x1, x2, W_self0, W_neigh0, b0, W_self1, W_neigh1, b1,
           src0, dst0, src1, dst1, rb2):
    split = _split_call(dst0)
    pad = jnp.zeros((SB,), jnp.int32)
    src0p = jnp.concatenate([src0, pad])
    dst0p = jnp.concatenate([dst0, pad])
    zrows = jnp.zeros((N2 // NS, D), jnp.float32)
    onesw = jnp.ones((B, CW), jnp.float32)

    seg = _sc_layer0(x0, src0p, dst0p, split, zrows)
    cnt = _sc_cnt0(dst0p, split, zrows, onesw)
    h1 = _tc_layer0(
        x1.reshape(2, N1 // 2, D),
        seg.reshape(2, ACC0, D),
        cnt.reshape(2, ACC0, CW),
        W_self0, W_neigh0, b0.reshape(1, D),
    ).reshape(N1, D)

    part, cntp = _sc_layer1(h1, src1, dst1, zrows, onesw)
    hd = lax.dynamic_slice_in_dim(h1, rb2 - N2, N2)
    ws1p = jnp.pad(W_self1, ((0, 0), (0, D - 40)))
    wn1p = jnp.pad(W_neigh1, ((0, 0), (0, D - 40)))
    b1p = jnp.pad(b1, (0, D - 40)).reshape(1, D)
    outp = _tc_layer1(hd, part, cntp, ws1p, wn1p, b1p)
    return outp[:, :40]
